# baseline (speedup 1.0000x reference)
import functools

import jax
import jax.numpy as jnp
import numpy as np
from jax import lax
from jax.experimental import pallas as pl
from jax.experimental.pallas import tpu as pltpu
from jax.experimental.pallas import tpu_sc as plsc

N = 10000
E = 160000
D = 256
DQ = 64
G = 8
NC = 2
NS = 16
L = 16
NP = 10240
RPT = NP // NS
NR = NP // 128
EPT = NP
EPAD = NS * EPT
CH = 128
NCHUNK = EPT // CH
CQ = 32
NQCHUNK = EPT // CQ
NSL = 12
GDEPTH = 9
SRT = RPT // 128
RPT_LAST = N - (NS - 1) * RPT

_GRID = np.linspace(-2.0, 2.0, G).astype(np.float32)
_INV_DENOM = np.float32(1.0 / ((_GRID[-1] - _GRID[0]) / (G - 1)))

_BM = 1000


def _dense_body(h_ref, bw_ref, sw_ref, w_ref, at_ref,
                k0_ref, k1_ref, k2_ref, k3_ref, s1_ref, s2_ref):
    h = h_ref[...]
    acc = jnp.dot(h * jax.nn.sigmoid(h), bw_ref[...],
                  preferred_element_type=jnp.float32)
    for g in range(G):
        z = (h - _GRID[g]) * _INV_DENOM
        phi = jnp.exp(-(z * z))
        acc = acc + jnp.dot(phi, sw_ref[g],
                            preferred_element_type=jnp.float32)
    k0_ref[...] = acc[:, 0 * DQ:1 * DQ]
    k1_ref[...] = acc[:, 1 * DQ:2 * DQ]
    k2_ref[...] = acc[:, 2 * DQ:3 * DQ]
    k3_ref[...] = acc[:, 3 * DQ:4 * DQ]
    a1 = at_ref[:, :D]
    a2 = at_ref[:, D:]
    wa1 = jnp.sum(w_ref[...] * a1, axis=1)
    wa2 = jnp.sum(w_ref[...] * a2, axis=1)
    s1 = jnp.sum(h * wa1[None, :], axis=1, keepdims=True)
    s2 = jnp.sum(h * wa2[None, :], axis=1, keepdims=True)
    s1_ref[...] = jnp.broadcast_to(s1, (h.shape[0], DQ))
    s2_ref[...] = jnp.broadcast_to(s2, (h.shape[0], DQ))


def _dense(h, base_weight, sw_r, W, aT):
    nblk = N // _BM
    qspec = pl.BlockSpec((_BM, DQ), lambda i: (i, 0))
    qshape = jax.ShapeDtypeStruct((N, DQ), jnp.float32)
    return pl.pallas_call(
        _dense_body,
        grid=(nblk,),
        in_specs=[
            pl.BlockSpec((_BM, D), lambda i: (i, 0)),
            pl.BlockSpec((D, D), lambda i: (0, 0)),
            pl.BlockSpec((G, D, D), lambda i: (0, 0, 0)),
            pl.BlockSpec((D, D), lambda i: (0, 0)),
            pl.BlockSpec((1, 2 * D), lambda i: (0, 0)),
        ],
        out_specs=[qspec] * 6,
        out_shape=[qshape] * 6,
    )(h, base_weight, sw_r, W, aT)


def _split_idx(i16):
    return [lax.shift_right_logical(i16, 7), lax.bitwise_and(i16, 127)]


def _sc_body(s1_hbm, s2_hbm, rowsq_hbm, colsq_hbm,
             k0, k1, k2, k3,
             out,
             vs1, vs2, vqrows, vqcols, ve, vss, vidx,
             rb,
             vm16, vmax_all,
             acc, maxstage, ss_final, gsem, ssem):
    c = lax.axis_index("c")
    s = lax.axis_index("s")

    pltpu.sync_copy(s1_hbm, vs1)
    pltpu.sync_copy(s2_hbm, vs2)
    pltpu.sync_copy(rowsq_hbm.at[s], vqrows)
    pltpu.sync_copy(colsq_hbm.at[s], vqcols)

    def _ve_idx(i, g):
        flat16 = (jnp.full((L,), i * CQ + g * L, jnp.int32)
                  + lax.iota(jnp.int32, L))
        return _split_idx(flat16)

    @plsc.parallel_loop(0, NQCHUNK, unroll=4,
                        carry=jnp.full((L,), -1e30, jnp.float32))
    def _e_loop(i, m):
        for g in range(CQ // L):
            r16 = vqrows[i, pl.ds(g * L, L)]
            c16 = vqcols[i, pl.ds(g * L, L)]
            sg = (plsc.load_gather(vs1, _split_idx(r16))
                  + plsc.load_gather(vs2, _split_idx(c16)))
            e16 = jnp.maximum(sg, 0.2 * sg)
            plsc.store_scatter(ve, _ve_idx(i, g), e16)
            m = jnp.maximum(m, e16)
        return m

    vm16[...] = _e_loop
    pltpu.sync_copy(vm16, maxstage.at[s])
    plsc.subcore_barrier()
    pltpu.sync_copy(maxstage, vmax_all)
    mm = vmax_all[0, :]
    for t in range(1, NS):
        mm = jnp.maximum(mm, vmax_all[t, :])
    gmax = jnp.max(mm)

    for gg in range(NR // L):
        vidx[pl.ds(gg * L, L)] = lax.iota(jnp.int32, L) + gg * L

    def z_step(q, _):
        for g in range(128 // L):
            vss[q, pl.ds(g * L, L)] = jnp.zeros((L,), jnp.float32)
        return 0

    lax.fori_loop(0, NR, z_step, 0)

    @pl.when(s == 0)
    def _():
        pltpu.sync_copy(vss, ss_final)
    plsc.subcore_barrier()

    @plsc.parallel_loop(0, NQCHUNK, unroll=4)
    def _ex_loop(i):
        for g in range(CQ // L):
            r16 = vqrows[i, pl.ds(g * L, L)]
            vix = _ve_idx(i, g)
            ex16 = jnp.exp(plsc.load_gather(ve, vix) - gmax)
            plsc.store_scatter(ve, vix, ex16)
            plsc.addupdate_scatter(vss, _split_idx(r16), ex16)

    pltpu.sync_copy(vss, ss_final.at[vidx], add=True)
    plsc.subcore_barrier()
    pltpu.sync_copy(ss_final, vss)

    @plsc.parallel_loop(0, NQCHUNK, unroll=4)
    def _att_loop(i):
        for g in range(CQ // L):
            r16 = vqrows[i, pl.ds(g * L, L)]
            vix = _ve_idx(i, g)
            ss16 = plsc.load_gather(vss, _split_idx(r16))
            plsc.store_scatter(ve, vix,
                               plsc.load_gather(ve, vix) / (ss16 + 1e-16))

    def zb_step(j, _):
        for g in range(DQ // L):
            rb[j, pl.ds(g * L, L)] = jnp.zeros((L,), jnp.float32)
        return 0

    def agg_pass(kan_q, q):
        lax.fori_loop(0, CH, zb_step, 0)
        for b in range(RPT // CH):
            pltpu.sync_copy(rb.at[pl.ds(0, CH)],
                            acc.at[pl.ds(s * RPT + b * CH, CH)])
        plsc.subcore_barrier()

        def slot(sl):
            return rb.at[pl.ds(sl * CQ, CQ)]

        def issue_g(j):
            pltpu.async_copy(kan_q.at[vqcols.at[j]],
                             slot(lax.rem(j, NSL)), gsem)

        def drain_g():
            pltpu.make_async_copy(kan_q.at[vqcols.at[0]], slot(0),
                                  gsem).wait()

        def issue_s(j):
            pltpu.async_copy(slot(lax.rem(j, NSL)),
                             acc.at[vqrows.at[j]], ssem, add=True)

        def drain_s():
            pltpu.make_async_copy(slot(0), acc.at[vqrows.at[0]],
                                  ssem).wait()

        def scale(j):
            base = lax.rem(j, NSL) * CQ

            @plsc.parallel_loop(0, CQ, unroll=8)
            def _(jj):
                flat = j * CQ + jj
                att = plsc.load_gather(
                    ve, [jnp.full((L,), lax.shift_right_logical(flat, 7),
                                  jnp.int32),
                         jnp.full((L,), lax.bitwise_and(flat, 127),
                                  jnp.int32)])
                for g in range(DQ // L):
                    rb[base + jj, pl.ds(g * L, L)] = (
                        rb[base + jj, pl.ds(g * L, L)] * att)

        def prime(j, _):
            issue_g(j)
            return 0

        lax.fori_loop(0, GDEPTH, prime, 0)

        def pipe_step(j, _):
            @pl.when(j >= 2)
            def _():
                drain_s()

            @pl.when(j + GDEPTH < NQCHUNK)
            def _():
                issue_g(j + GDEPTH)

            drain_g()
            scale(j)
            issue_s(j)
            return 0

        lax.fori_loop(0, NQCHUNK, pipe_step, 0)
        drain_s()
        drain_s()
        plsc.subcore_barrier()
        col = (2 * c + q) * DQ

        @pl.when(s < NS - 1)
        def _():
            pltpu.sync_copy(acc.at[pl.ds(s * RPT, RPT)],
                            out.at[pl.ds(s * RPT, RPT), pl.ds(col, DQ)])

        @pl.when(s == NS - 1)
        def _():
            pltpu.sync_copy(acc.at[pl.ds(s * RPT, RPT_LAST)],
                            out.at[pl.ds(s * RPT, RPT_LAST),
                                   pl.ds(col, DQ)])
        plsc.subcore_barrier()

    @pl.when(c == 0)
    def _():
        agg_pass(k0, 0)
        agg_pass(k1, 1)

    @pl.when(c == 1)
    def _():
        agg_pass(k2, 0)
        agg_pass(k3, 1)


_sc_call = functools.partial(
    pl.kernel,
    mesh=plsc.VectorSubcoreMesh(core_axis_name="c", subcore_axis_name="s"),
    compiler_params=pltpu.CompilerParams(needs_layout_passes=False,
                                         use_tc_tiling_on_sc=False),
    out_type=jax.ShapeDtypeStruct((N, D), jnp.float32),
    scratch_types=[
        pltpu.VMEM((NR, 128), jnp.float32),
        pltpu.VMEM((NR, 128), jnp.float32),
        pltpu.VMEM((NQCHUNK, CQ), jnp.int32),
        pltpu.VMEM((NQCHUNK, CQ), jnp.int32),
        pltpu.VMEM((NCHUNK, CH), jnp.float32),
        pltpu.VMEM((NR, 128), jnp.float32),
        pltpu.VMEM((NR,), jnp.int32),
        pltpu.VMEM((NSL * CQ, DQ), jnp.float32),
        pltpu.VMEM((L,), jnp.float32),
        pltpu.VMEM((NS, L), jnp.float32),
        pltpu.VMEM_SHARED((NP, DQ), jnp.float32),
        pltpu.VMEM_SHARED((NS, L), jnp.float32),
        pltpu.VMEM_SHARED((NR, 128), jnp.float32),
        pltpu.SemaphoreType.DMA,
        pltpu.SemaphoreType.DMA,
    ],
)(_sc_body)


def kernel(h, edge_index, W, a, base_weight, spline_weight):
    aT = a.reshape(1, 2 * D)
    sw_r = spline_weight.reshape(D, G, D).transpose(1, 0, 2)
    k0, k1, k2, k3, s1b, s2b = _dense(h, base_weight, sw_r, W, aT)
    npad = EPAD - E
    s1 = jnp.concatenate([s1b[:, 0], jnp.full((NP - N,), -1e30, jnp.float32)])
    s2 = jnp.concatenate([s2b[:, 0], jnp.zeros((NP - N,), jnp.float32)])
    rowsq = jnp.concatenate(
        [edge_index[0], jnp.full((npad,), NP - 1, jnp.int32)])
    colsq = jnp.concatenate(
        [edge_index[1], jnp.zeros((npad,), jnp.int32)])
    return _sc_call(s1.reshape(NR, 128), s2.reshape(NR, 128),
                    rowsq.reshape(NS, NQCHUNK, CQ),
                    colsq.reshape(NS, NQCHUNK, CQ),
                    k0, k1, k2, k3)


# speedup vs baseline: 1.0058x; 1.0058x over previous
import functools

import jax
import jax.numpy as jnp
import numpy as np
from jax import lax
from jax.experimental import pallas as pl
from jax.experimental.pallas import tpu as pltpu
from jax.experimental.pallas import tpu_sc as plsc

N = 10000
E = 160000
D = 256
DQ = 64
G = 8
NC = 2
NS = 16
L = 16
NP = 10240
RPT = NP // NS
NR = NP // 128
EPT = NP
EPAD = NS * EPT
CH = 128
NCHUNK = EPT // CH
CQ = 32
NQCHUNK = EPT // CQ
NSL = 12
GDEPTH = 9
SRT = RPT // 128
RPT_LAST = N - (NS - 1) * RPT

_GRID = np.linspace(-2.0, 2.0, G).astype(np.float32)
_INV_DENOM = np.float32(1.0 / ((_GRID[-1] - _GRID[0]) / (G - 1)))

_BM = 1000


def _dense_body(h_ref, bw_ref, sw_ref, w_ref, at_ref,
                k0_ref, k1_ref, k2_ref, k3_ref, s1_ref, s2_ref):
    h = h_ref[...]
    acc = jnp.dot(h * jax.nn.sigmoid(h), bw_ref[...],
                  preferred_element_type=jnp.float32)
    for g in range(G):
        z = (h - _GRID[g]) * _INV_DENOM
        phi = jnp.exp(-(z * z))
        acc = acc + jnp.dot(phi, sw_ref[g],
                            preferred_element_type=jnp.float32)
    k0_ref[...] = acc[:, 0 * DQ:1 * DQ]
    k1_ref[...] = acc[:, 1 * DQ:2 * DQ]
    k2_ref[...] = acc[:, 2 * DQ:3 * DQ]
    k3_ref[...] = acc[:, 3 * DQ:4 * DQ]
    a1 = at_ref[:, :D]
    a2 = at_ref[:, D:]
    wa1 = jnp.sum(w_ref[...] * a1, axis=1)
    wa2 = jnp.sum(w_ref[...] * a2, axis=1)
    s1 = jnp.sum(h * wa1[None, :], axis=1, keepdims=True)
    s2 = jnp.sum(h * wa2[None, :], axis=1, keepdims=True)
    s1_ref[...] = jnp.broadcast_to(s1, (h.shape[0], DQ))
    s2_ref[...] = jnp.broadcast_to(s2, (h.shape[0], DQ))


def _dense(h, base_weight, sw_r, W, aT):
    nblk = N // _BM
    qspec = pl.BlockSpec((_BM, DQ), lambda i: (i, 0))
    qshape = jax.ShapeDtypeStruct((N, DQ), jnp.float32)
    return pl.pallas_call(
        _dense_body,
        grid=(nblk,),
        in_specs=[
            pl.BlockSpec((_BM, D), lambda i: (i, 0)),
            pl.BlockSpec((D, D), lambda i: (0, 0)),
            pl.BlockSpec((G, D, D), lambda i: (0, 0, 0)),
            pl.BlockSpec((D, D), lambda i: (0, 0)),
            pl.BlockSpec((1, 2 * D), lambda i: (0, 0)),
        ],
        out_specs=[qspec] * 6,
        out_shape=[qshape] * 6,
    )(h, base_weight, sw_r, W, aT)


def _split_idx(i16):
    return [lax.shift_right_logical(i16, 7), lax.bitwise_and(i16, 127)]


def _sc_body(s1_hbm, s2_hbm, rowsq_hbm, colsq_hbm,
             k0, k1, k2, k3,
             out,
             vs1, vs2, vqrows, vqcols, ve, vss, vidx,
             rb,
             vm16, vmax_all,
             acc, maxstage, ss_final, gsem, ssem):
    c = lax.axis_index("c")
    s = lax.axis_index("s")

    pltpu.sync_copy(s1_hbm, vs1)
    pltpu.sync_copy(s2_hbm, vs2)
    pltpu.sync_copy(rowsq_hbm.at[s], vqrows)
    pltpu.sync_copy(colsq_hbm.at[s], vqcols)

    def _ve_idx(i, g):
        flat16 = (jnp.full((L,), i * CQ + g * L, jnp.int32)
                  + lax.iota(jnp.int32, L))
        return _split_idx(flat16)

    @plsc.parallel_loop(0, NQCHUNK, unroll=2,
                        carry=jnp.full((L,), -1e30, jnp.float32))
    def _e_loop(i, m):
        for g in range(CQ // L):
            r16 = vqrows[i, pl.ds(g * L, L)]
            c16 = vqcols[i, pl.ds(g * L, L)]
            sg = (plsc.load_gather(vs1, _split_idx(r16))
                  + plsc.load_gather(vs2, _split_idx(c16)))
            e16 = jnp.maximum(sg, 0.2 * sg)
            plsc.store_scatter(ve, _ve_idx(i, g), e16)
            m = jnp.maximum(m, e16)
        return m

    vm16[...] = _e_loop
    pltpu.sync_copy(vm16, maxstage.at[s])
    plsc.subcore_barrier()
    pltpu.sync_copy(maxstage, vmax_all)
    mm = vmax_all[0, :]
    for t in range(1, NS):
        mm = jnp.maximum(mm, vmax_all[t, :])
    gmax = jnp.max(mm)

    for gg in range(NR // L):
        vidx[pl.ds(gg * L, L)] = lax.iota(jnp.int32, L) + gg * L

    def z_step(q, _):
        for g in range(128 // L):
            vss[q, pl.ds(g * L, L)] = jnp.zeros((L,), jnp.float32)
        return 0

    lax.fori_loop(0, NR, z_step, 0)

    @pl.when(s == 0)
    def _():
        pltpu.sync_copy(vss, ss_final)
    plsc.subcore_barrier()

    @plsc.parallel_loop(0, NQCHUNK, unroll=2)
    def _ex_loop(i):
        for g in range(CQ // L):
            r16 = vqrows[i, pl.ds(g * L, L)]
            vix = _ve_idx(i, g)
            ex16 = jnp.exp(plsc.load_gather(ve, vix) - gmax)
            plsc.store_scatter(ve, vix, ex16)
            plsc.addupdate_scatter(vss, _split_idx(r16), ex16)

    pltpu.sync_copy(vss, ss_final.at[vidx], add=True)
    plsc.subcore_barrier()
    pltpu.sync_copy(ss_final, vss)

    @plsc.parallel_loop(0, NQCHUNK, unroll=2)
    def _att_loop(i):
        for g in range(CQ // L):
            r16 = vqrows[i, pl.ds(g * L, L)]
            vix = _ve_idx(i, g)
            ss16 = plsc.load_gather(vss, _split_idx(r16))
            plsc.store_scatter(ve, vix,
                               plsc.load_gather(ve, vix) / (ss16 + 1e-16))

    def zb_step(j, _):
        for g in range(DQ // L):
            rb[j, pl.ds(g * L, L)] = jnp.zeros((L,), jnp.float32)
        return 0

    def agg_pass(kan_q, q):
        lax.fori_loop(0, CH, zb_step, 0)
        for b in range(RPT // CH):
            pltpu.sync_copy(rb.at[pl.ds(0, CH)],
                            acc.at[pl.ds(s * RPT + b * CH, CH)])
        plsc.subcore_barrier()

        def slot(sl):
            return rb.at[pl.ds(sl * CQ, CQ)]

        def issue_g(j):
            pltpu.async_copy(kan_q.at[vqcols.at[j]],
                             slot(lax.rem(j, NSL)), gsem)

        def drain_g():
            pltpu.make_async_copy(kan_q.at[vqcols.at[0]], slot(0),
                                  gsem).wait()

        def issue_s(j):
            pltpu.async_copy(slot(lax.rem(j, NSL)),
                             acc.at[vqrows.at[j]], ssem, add=True)

        def drain_s():
            pltpu.make_async_copy(slot(0), acc.at[vqrows.at[0]],
                                  ssem).wait()

        def scale(j):
            base = lax.rem(j, NSL) * CQ

            @plsc.parallel_loop(0, CQ, unroll=4)
            def _(jj):
                flat = j * CQ + jj
                att = plsc.load_gather(
                    ve, [jnp.full((L,), lax.shift_right_logical(flat, 7),
                                  jnp.int32),
                         jnp.full((L,), lax.bitwise_and(flat, 127),
                                  jnp.int32)])
                for g in range(DQ // L):
                    rb[base + jj, pl.ds(g * L, L)] = (
                        rb[base + jj, pl.ds(g * L, L)] * att)

        def prime(j, _):
            issue_g(j)
            return 0

        lax.fori_loop(0, GDEPTH, prime, 0)

        def pipe_step(j, _):
            @pl.when(j >= 2)
            def _():
                drain_s()

            @pl.when(j + GDEPTH < NQCHUNK)
            def _():
                issue_g(j + GDEPTH)

            drain_g()
            scale(j)
            issue_s(j)
            return 0

        lax.fori_loop(0, NQCHUNK, pipe_step, 0)
        drain_s()
        drain_s()
        plsc.subcore_barrier()
        col = (2 * c + q) * DQ

        @pl.when(s < NS - 1)
        def _():
            pltpu.sync_copy(acc.at[pl.ds(s * RPT, RPT)],
                            out.at[pl.ds(s * RPT, RPT), pl.ds(col, DQ)])

        @pl.when(s == NS - 1)
        def _():
            pltpu.sync_copy(acc.at[pl.ds(s * RPT, RPT_LAST)],
                            out.at[pl.ds(s * RPT, RPT_LAST),
                                   pl.ds(col, DQ)])
        plsc.subcore_barrier()

    @pl.when(c == 0)
    def _():
        agg_pass(k0, 0)
        agg_pass(k1, 1)

    @pl.when(c == 1)
    def _():
        agg_pass(k2, 0)
        agg_pass(k3, 1)


_sc_call = functools.partial(
    pl.kernel,
    mesh=plsc.VectorSubcoreMesh(core_axis_name="c", subcore_axis_name="s"),
    compiler_params=pltpu.CompilerParams(needs_layout_passes=False,
                                         use_tc_tiling_on_sc=False),
    out_type=jax.ShapeDtypeStruct((N, D), jnp.float32),
    scratch_types=[
        pltpu.VMEM((NR, 128), jnp.float32),
        pltpu.VMEM((NR, 128), jnp.float32),
        pltpu.VMEM((NQCHUNK, CQ), jnp.int32),
        pltpu.VMEM((NQCHUNK, CQ), jnp.int32),
        pltpu.VMEM((NCHUNK, CH), jnp.float32),
        pltpu.VMEM((NR, 128), jnp.float32),
        pltpu.VMEM((NR,), jnp.int32),
        pltpu.VMEM((NSL * CQ, DQ), jnp.float32),
        pltpu.VMEM((L,), jnp.float32),
        pltpu.VMEM((NS, L), jnp.float32),
        pltpu.VMEM_SHARED((NP, DQ), jnp.float32),
        pltpu.VMEM_SHARED((NS, L), jnp.float32),
        pltpu.VMEM_SHARED((NR, 128), jnp.float32),
        pltpu.SemaphoreType.DMA,
        pltpu.SemaphoreType.DMA,
    ],
)(_sc_body)


def kernel(h, edge_index, W, a, base_weight, spline_weight):
    aT = a.reshape(1, 2 * D)
    sw_r = spline_weight.reshape(D, G, D).transpose(1, 0, 2)
    k0, k1, k2, k3, s1b, s2b = _dense(h, base_weight, sw_r, W, aT)
    npad = EPAD - E
    s1 = jnp.concatenate([s1b[:, 0], jnp.full((NP - N,), -1e30, jnp.float32)])
    s2 = jnp.concatenate([s2b[:, 0], jnp.zeros((NP - N,), jnp.float32)])
    rowsq = jnp.concatenate(
        [edge_index[0], jnp.full((npad,), NP - 1, jnp.int32)])
    colsq = jnp.concatenate(
        [edge_index[1], jnp.zeros((npad,), jnp.int32)])
    return _sc_call(s1.reshape(NR, 128), s2.reshape(NR, 128),
                    rowsq.reshape(NS, NQCHUNK, CQ),
                    colsq.reshape(NS, NQCHUNK, CQ),
                    k0, k1, k2, k3)


# speedup vs baseline: 1.4401x; 1.4318x over previous
import functools

import jax
import jax.numpy as jnp
import numpy as np
from jax import lax
from jax.experimental import pallas as pl
from jax.experimental.pallas import tpu as pltpu
from jax.experimental.pallas import tpu_sc as plsc

N = 10000
E = 160000
D = 256
DQ = 64
G = 8
NC = 2
NS = 16
L = 16
NP = 10240
RPT = NP // NS
NR = NP // 128
EPT = NP
EPAD = NS * EPT
CH = 128
NCHUNK = EPT // CH
CQ = 32
NQCHUNK = EPT // CQ
NSL = 12
SSL = 4
GDEPTH = 9
SRT = RPT // 128
RPT_LAST = N - (NS - 1) * RPT

_GRID = np.linspace(-2.0, 2.0, G).astype(np.float32)
_INV_DENOM = np.float32(1.0 / ((_GRID[-1] - _GRID[0]) / (G - 1)))

_BM = 1000


def _dense_body(h_ref, bw_ref, sw_ref, w_ref, at_ref,
                k0_ref, k1_ref, k2_ref, k3_ref, s1_ref, s2_ref):
    h = h_ref[...]
    acc = jnp.dot(h * jax.nn.sigmoid(h), bw_ref[...],
                  preferred_element_type=jnp.float32)
    for g in range(G):
        z = (h - _GRID[g]) * _INV_DENOM
        phi = jnp.exp(-(z * z))
        acc = acc + jnp.dot(phi, sw_ref[g],
                            preferred_element_type=jnp.float32)
    k0_ref[...] = acc[:, 0 * DQ:1 * DQ].astype(jnp.bfloat16)
    k1_ref[...] = acc[:, 1 * DQ:2 * DQ].astype(jnp.bfloat16)
    k2_ref[...] = acc[:, 2 * DQ:3 * DQ].astype(jnp.bfloat16)
    k3_ref[...] = acc[:, 3 * DQ:4 * DQ].astype(jnp.bfloat16)
    a1 = at_ref[:, :D]
    a2 = at_ref[:, D:]
    wa1 = jnp.sum(w_ref[...] * a1, axis=1)
    wa2 = jnp.sum(w_ref[...] * a2, axis=1)
    s1 = jnp.sum(h * wa1[None, :], axis=1, keepdims=True)
    s2 = jnp.sum(h * wa2[None, :], axis=1, keepdims=True)
    s1_ref[...] = jnp.broadcast_to(s1, (h.shape[0], DQ))
    s2_ref[...] = jnp.broadcast_to(s2, (h.shape[0], DQ))


def _dense(h, base_weight, sw_r, W, aT):
    nblk = N // _BM
    qspec = pl.BlockSpec((_BM, DQ), lambda i: (i, 0))
    qshape_b = jax.ShapeDtypeStruct((N, DQ), jnp.bfloat16)
    qshape = jax.ShapeDtypeStruct((N, DQ), jnp.float32)
    return pl.pallas_call(
        _dense_body,
        grid=(nblk,),
        in_specs=[
            pl.BlockSpec((_BM, D), lambda i: (i, 0)),
            pl.BlockSpec((D, D), lambda i: (0, 0)),
            pl.BlockSpec((G, D, D), lambda i: (0, 0, 0)),
            pl.BlockSpec((D, D), lambda i: (0, 0)),
            pl.BlockSpec((1, 2 * D), lambda i: (0, 0)),
        ],
        out_specs=[qspec] * 6,
        out_shape=[qshape_b] * 4 + [qshape] * 2,
    )(h, base_weight, sw_r, W, aT)


def _split_idx(i16):
    return [lax.shift_right_logical(i16, 7), lax.bitwise_and(i16, 127)]


def _sc_body(s1_hbm, s2_hbm, rowsq_hbm, colsq_hbm,
             k0, k1, k2, k3,
             out,
             vs1, vs2, vqrows, vqcols, ve, vss, vidx,
             rbg, rbs,
             vm16, vmax_all,
             acc, maxstage, ss_final, gsem, ssem):
    c = lax.axis_index("c")
    s = lax.axis_index("s")

    pltpu.sync_copy(s1_hbm, vs1)
    pltpu.sync_copy(s2_hbm, vs2)
    pltpu.sync_copy(rowsq_hbm.at[s], vqrows)
    pltpu.sync_copy(colsq_hbm.at[s], vqcols)

    def _ve_idx(i, g):
        flat16 = (jnp.full((L,), i * CQ + g * L, jnp.int32)
                  + lax.iota(jnp.int32, L))
        return _split_idx(flat16)

    @plsc.parallel_loop(0, NQCHUNK, unroll=2,
                        carry=jnp.full((L,), -1e30, jnp.float32))
    def _e_loop(i, m):
        for g in range(CQ // L):
            r16 = vqrows[i, pl.ds(g * L, L)]
            c16 = vqcols[i, pl.ds(g * L, L)]
            sg = (plsc.load_gather(vs1, _split_idx(r16))
                  + plsc.load_gather(vs2, _split_idx(c16)))
            e16 = jnp.maximum(sg, 0.2 * sg)
            plsc.store_scatter(ve, _ve_idx(i, g), e16)
            m = jnp.maximum(m, e16)
        return m

    vm16[...] = _e_loop
    pltpu.sync_copy(vm16, maxstage.at[s])
    plsc.subcore_barrier()
    pltpu.sync_copy(maxstage, vmax_all)
    mm = vmax_all[0, :]
    for t in range(1, NS):
        mm = jnp.maximum(mm, vmax_all[t, :])
    gmax = jnp.max(mm)

    for gg in range(NR // L):
        vidx[pl.ds(gg * L, L)] = lax.iota(jnp.int32, L) + gg * L

    def z_step(q, _):
        for g in range(128 // L):
            vss[q, pl.ds(g * L, L)] = jnp.zeros((L,), jnp.float32)
        return 0

    lax.fori_loop(0, NR, z_step, 0)

    @pl.when(s == 0)
    def _():
        pltpu.sync_copy(vss, ss_final)
    plsc.subcore_barrier()

    @plsc.parallel_loop(0, NQCHUNK, unroll=2)
    def _ex_loop(i):
        for g in range(CQ // L):
            r16 = vqrows[i, pl.ds(g * L, L)]
            vix = _ve_idx(i, g)
            ex16 = jnp.exp(plsc.load_gather(ve, vix) - gmax)
            plsc.store_scatter(ve, vix, ex16)
            plsc.addupdate_scatter(vss, _split_idx(r16), ex16)

    pltpu.sync_copy(vss, ss_final.at[vidx], add=True)
    plsc.subcore_barrier()
    pltpu.sync_copy(ss_final, vss)

    @plsc.parallel_loop(0, NQCHUNK, unroll=2)
    def _att_loop(i):
        for g in range(CQ // L):
            r16 = vqrows[i, pl.ds(g * L, L)]
            vix = _ve_idx(i, g)
            ss16 = plsc.load_gather(vss, _split_idx(r16))
            plsc.store_scatter(ve, vix,
                               plsc.load_gather(ve, vix) / (ss16 + 1e-16))

    def zb_step(j, _):
        for g in range(DQ // L):
            rbs[j, pl.ds(g * L, L)] = jnp.zeros((L,), jnp.float32)
        return 0

    def agg_pass(kan_q, q):
        lax.fori_loop(0, SSL * CQ, zb_step, 0)
        for b in range(RPT // (SSL * CQ)):
            pltpu.sync_copy(rbs.at[pl.ds(0, SSL * CQ)],
                            acc.at[pl.ds(s * RPT + b * SSL * CQ,
                                         SSL * CQ)])
        plsc.subcore_barrier()

        def slot_g(sl):
            return rbg.at[pl.ds(sl * CQ, CQ)]

        def slot_s(sl):
            return rbs.at[pl.ds(sl * CQ, CQ)]

        def issue_g(j):
            pltpu.async_copy(kan_q.at[vqcols.at[j]],
                             slot_g(lax.rem(j, NSL)), gsem)

        def drain_g():
            pltpu.make_async_copy(kan_q.at[vqcols.at[0]], slot_g(0),
                                  gsem).wait()

        def issue_s(j):
            pltpu.async_copy(slot_s(lax.rem(j, SSL)),
                             acc.at[vqrows.at[j]], ssem, add=True)

        def drain_s():
            pltpu.make_async_copy(slot_s(0), acc.at[vqrows.at[0]],
                                  ssem).wait()

        col_a = 2 * lax.iota(jnp.int32, L)
        col_b = col_a + 1

        def scale(j):
            gbase = lax.rem(j, NSL) * CQ
            sbase = lax.rem(j, SSL) * CQ

            @plsc.parallel_loop(0, CQ, unroll=4)
            def _(jj):
                flat = j * CQ + jj
                att = plsc.load_gather(
                    ve, [jnp.full((L,), lax.shift_right_logical(flat, 7),
                                  jnp.int32),
                         jnp.full((L,), lax.bitwise_and(flat, 127),
                                  jnp.int32)])
                srow = jnp.full((L,), sbase + jj, jnp.int32)
                for g in range(DQ // (2 * L)):
                    x = rbg[gbase + jj, pl.ds(g * 2 * L, 2 * L)]
                    a, b = plsc.unpack(x, format=plsc.PackFormat.INTERLEAVED,
                                       preferred_element_type=jnp.float32)
                    plsc.store_scatter(rbs, [srow, col_a + g * 2 * L],
                                       a * att)
                    plsc.store_scatter(rbs, [srow, col_b + g * 2 * L],
                                       b * att)

        def prime(j, _):
            issue_g(j)
            return 0

        lax.fori_loop(0, GDEPTH, prime, 0)

        def pipe_step(j, _):
            @pl.when(j >= 2)
            def _():
                drain_s()

            @pl.when(j + GDEPTH < NQCHUNK)
            def _():
                issue_g(j + GDEPTH)

            drain_g()
            scale(j)
            issue_s(j)
            return 0

        lax.fori_loop(0, NQCHUNK, pipe_step, 0)
        drain_s()
        drain_s()
        plsc.subcore_barrier()
        col = (2 * c + q) * DQ

        @pl.when(s < NS - 1)
        def _():
            pltpu.sync_copy(acc.at[pl.ds(s * RPT, RPT)],
                            out.at[pl.ds(s * RPT, RPT), pl.ds(col, DQ)])

        @pl.when(s == NS - 1)
        def _():
            pltpu.sync_copy(acc.at[pl.ds(s * RPT, RPT_LAST)],
                            out.at[pl.ds(s * RPT, RPT_LAST),
                                   pl.ds(col, DQ)])
        plsc.subcore_barrier()

    @pl.when(c == 0)
    def _():
        agg_pass(k0, 0)
        agg_pass(k1, 1)

    @pl.when(c == 1)
    def _():
        agg_pass(k2, 0)
        agg_pass(k3, 1)


_sc_call = functools.partial(
    pl.kernel,
    mesh=plsc.VectorSubcoreMesh(core_axis_name="c", subcore_axis_name="s"),
    compiler_params=pltpu.CompilerParams(needs_layout_passes=False,
                                         use_tc_tiling_on_sc=False),
    out_type=jax.ShapeDtypeStruct((N, D), jnp.float32),
    scratch_types=[
        pltpu.VMEM((NR, 128), jnp.float32),
        pltpu.VMEM((NR, 128), jnp.float32),
        pltpu.VMEM((NQCHUNK, CQ), jnp.int32),
        pltpu.VMEM((NQCHUNK, CQ), jnp.int32),
        pltpu.VMEM((NCHUNK, CH), jnp.float32),
        pltpu.VMEM((NR, 128), jnp.float32),
        pltpu.VMEM((NR,), jnp.int32),
        pltpu.VMEM((NSL * CQ, DQ), jnp.bfloat16),
        pltpu.VMEM((SSL * CQ, DQ), jnp.float32),
        pltpu.VMEM((L,), jnp.float32),
        pltpu.VMEM((NS, L), jnp.float32),
        pltpu.VMEM_SHARED((NP, DQ), jnp.float32),
        pltpu.VMEM_SHARED((NS, L), jnp.float32),
        pltpu.VMEM_SHARED((NR, 128), jnp.float32),
        pltpu.SemaphoreType.DMA,
        pltpu.SemaphoreType.DMA,
    ],
)(_sc_body)


def kernel(h, edge_index, W, a, base_weight, spline_weight):
    aT = a.reshape(1, 2 * D)
    sw_r = spline_weight.reshape(D, G, D).transpose(1, 0, 2)
    k0, k1, k2, k3, s1b, s2b = _dense(h, base_weight, sw_r, W, aT)
    npad = EPAD - E
    s1 = jnp.concatenate([s1b[:, 0], jnp.full((NP - N,), -1e30, jnp.float32)])
    s2 = jnp.concatenate([s2b[:, 0], jnp.zeros((NP - N,), jnp.float32)])
    rowsq = jnp.concatenate(
        [edge_index[0], jnp.full((npad,), NP - 1, jnp.int32)])
    colsq = jnp.concatenate(
        [edge_index[1], jnp.zeros((npad,), jnp.int32)])
    return _sc_call(s1.reshape(NR, 128), s2.reshape(NR, 128),
                    rowsq.reshape(NS, NQCHUNK, CQ),
                    colsq.reshape(NS, NQCHUNK, CQ),
                    k0, k1, k2, k3)
